# baseline (speedup 1.0000x reference)
import functools

import jax
import jax.numpy as jnp
from jax import lax
from jax.experimental import pallas as pl
from jax.experimental.pallas import tpu as pltpu
from jax.experimental.pallas import tpu_sc as plsc

NUM_ITEMS = 1000000
HIDDEN = 64
BATCH = 16384
NC, NS = 2, 16
NW = NC * NS
BPW = BATCH // NW
CHUNK = 128
NCH = BPW // CHUNK


def _gather_sc(idx, pflat, cflat):
    mesh = plsc.VectorSubcoreMesh(core_axis_name="c", subcore_axis_name="s")

    @functools.partial(
        pl.kernel,
        mesh=mesh,
        compiler_params=pltpu.CompilerParams(use_tc_tiling_on_sc=False),
        out_type=(
            jax.ShapeDtypeStruct((HIDDEN, BATCH), jnp.float32),
            jax.ShapeDtypeStruct((HIDDEN, BATCH), jnp.float32),
        ),
        scratch_types=[
            pltpu.VMEM((NCH, CHUNK), jnp.int32),
            pltpu.VMEM((NCH, CHUNK), jnp.int32),
            pltpu.VMEM((HIDDEN, BPW), jnp.float32),
            pltpu.VMEM((HIDDEN, BPW), jnp.float32),
            pltpu.SemaphoreType.DMA,
            pltpu.SemaphoreType.DMA,
        ],
    )
    def k(idx_hbm, p_hbm, c_hbm, p_out, c_out,
          idx_v, idxc_v, p_buf, c_buf, sem_p, sem_c):
        wid = lax.axis_index("c") * NS + lax.axis_index("s")
        base = wid * BPW
        for j in range(NCH):
            pltpu.sync_copy(idx_hbm.at[pl.ds(base + j * CHUNK, CHUNK)],
                            idx_v.at[j])

        def percol(c, carry):
            off = c * NUM_ITEMS
            for j in range(NCH):
                for r in range(CHUNK // 16):
                    s = pl.ds(r * 16, 16)
                    idxc_v[j, s] = idx_v[j, s] + off
            waits = []
            for j in range(NCH):
                waits.append(pltpu.async_copy(
                    p_hbm.at[idxc_v.at[j]], p_buf.at[c, pl.ds(j * CHUNK, CHUNK)],
                    sem_p))
                waits.append(pltpu.async_copy(
                    c_hbm.at[idxc_v.at[j]], c_buf.at[c, pl.ds(j * CHUNK, CHUNK)],
                    sem_c))
            for w in waits:
                w.wait()
            return carry

        lax.fori_loop(0, HIDDEN, percol, 0)
        pltpu.sync_copy(p_buf, p_out.at[:, pl.ds(base, BPW)])
        pltpu.sync_copy(c_buf, c_out.at[:, pl.ds(base, BPW)])

    return k(idx, pflat, cflat)


def _rating_tc(pt, ct, W, b):
    blk = 4096

    def body(p_ref, c_ref, w_ref, b_ref, o_ref):
        s = jnp.sum((p_ref[...] + c_ref[...]) * w_ref[...], axis=0, keepdims=True)
        o_ref[...] = jax.nn.sigmoid(s + b_ref[...])

    return pl.pallas_call(
        body,
        grid=(BATCH // blk,),
        in_specs=[
            pl.BlockSpec((HIDDEN, blk), lambda i: (0, i)),
            pl.BlockSpec((HIDDEN, blk), lambda i: (0, i)),
            pl.BlockSpec((HIDDEN, 1), lambda i: (0, 0)),
            pl.BlockSpec((1, 1), lambda i: (0, 0)),
        ],
        out_specs=pl.BlockSpec((1, blk), lambda i: (0, i)),
        out_shape=jax.ShapeDtypeStruct((1, BATCH), jnp.float32),
    )(pt, ct, W.reshape(HIDDEN, 1), b.reshape(1, 1))


def kernel(item_indices, item_personality_table, item_commonality_table, W, b):
    idx = item_indices.astype(jnp.int32)
    pflat = item_personality_table.T.reshape(-1)
    cflat = item_commonality_table.T.reshape(-1)
    pt_all, ct_all = _gather_sc(idx, pflat, cflat)
    rating = _rating_tc(pt_all, ct_all, W, b).reshape(BATCH, 1)
    return (rating, pt_all.T, ct_all.T)
